# baseline (speedup 1.0000x reference)
import jax
import jax.numpy as jnp
from jax.experimental import pallas as pl

_DBLK = 32


def _body(t_ref, o_ref):
    nhalf = pl.program_id(1)
    t = t_ref[...]
    c, h, w = o_ref.shape[1], o_ref.shape[2], o_ref.shape[3]

    @pl.when(nhalf < (128 // _DBLK))
    def _col():
        o_ref[...] = jnp.broadcast_to(t[None, :, None, :], (1, c, h, w))

    @pl.when(nhalf >= (128 // _DBLK))
    def _row():
        o_ref[...] = jnp.broadcast_to(t[None, :, :, None], (1, c, h, w))


def kernel(x, row_embed, col_embed):
    b = x.shape[0]
    h, w = x.shape[2], x.shape[3]
    d_half = row_embed.shape[1]
    d = 2 * d_half
    t = jnp.concatenate([col_embed[:w].T, row_embed[:h].T], axis=0)

    return pl.pallas_call(
        _body,
        grid=(b, d // _DBLK),
        in_specs=[pl.BlockSpec((_DBLK, t.shape[1]), lambda bb, dd: (dd, 0))],
        out_specs=pl.BlockSpec((1, _DBLK, h, w), lambda bb, dd: (bb, dd, 0, 0)),
        out_shape=jax.ShapeDtypeStruct((b, d, h, w), x.dtype),
    )(t)


# speedup vs baseline: 1.0000x; 1.0000x over previous
import jax
import jax.numpy as jnp
from jax.experimental import pallas as pl
from jax.experimental.pallas import tpu as pltpu

_DBLK = 32


def _body(t_ref, o_ref, s0, s1, sems):
    nblk = t_ref.shape[0] // _DBLK
    half = nblk // 2
    b = o_ref.shape[0]
    h, w = o_ref.shape[2], o_ref.shape[3]
    bufs = (s0, s1)

    def copies(blk):
        s = bufs[blk % 2]
        return [
            pltpu.make_async_copy(
                s, o_ref.at[bb, pl.ds(blk * _DBLK, _DBLK)], sems.at[blk % 2, bb]
            )
            for bb in range(b)
        ]

    for blk in range(nblk):
        s = bufs[blk % 2]
        if blk >= 2:
            for cp in copies(blk - 2):
                cp.wait()
        t = t_ref[pl.ds(blk * _DBLK, _DBLK), :]
        if blk < half:
            s[...] = jnp.broadcast_to(t[:, None, :], (_DBLK, h, w))
        else:
            s[...] = jnp.broadcast_to(t[:, :, None], (_DBLK, h, w))
        for cp in copies(blk):
            cp.start()
    for blk in (nblk - 2, nblk - 1):
        for cp in copies(blk):
            cp.wait()


def kernel(x, row_embed, col_embed):
    b = x.shape[0]
    h, w = x.shape[2], x.shape[3]
    d_half = row_embed.shape[1]
    d = 2 * d_half
    t = jnp.concatenate([col_embed[:w].T, row_embed[:h].T], axis=0)

    return pl.pallas_call(
        _body,
        in_specs=[pl.BlockSpec(memory_space=pltpu.VMEM)],
        out_specs=pl.BlockSpec(memory_space=pl.ANY),
        out_shape=jax.ShapeDtypeStruct((b, d, h, w), x.dtype),
        scratch_shapes=[
            pltpu.VMEM((_DBLK, h, w), jnp.float32),
            pltpu.VMEM((_DBLK, h, w), jnp.float32),
            pltpu.SemaphoreType.DMA((2, b)),
        ],
    )(t)
